# baseline (speedup 1.0000x reference)
import jax
import jax.numpy as jnp
from jax import lax
from jax.experimental import pallas as pl
from jax.experimental.pallas import tpu as pltpu
from jax.experimental.pallas import tpu_sc as plsc

_VOCAB = 100000
_D = 64
_S = 200
_B = 4096
_NC = 2
_NS = 16
_NW = _NC * _NS
_BT = _B // _NW
_ST = _S // 8
_NBUF = 4
_PDIST = _NBUF - 1
_NGROUPS = _S // _NBUF
_SCALE = 8.0
_KV = _D // 16


def _sc_body(tok_hbm, pe_hbm, table_hbm, out_hbm,
             tok_v, pe_v, r0, r1, r2, r3, o0, o1, o2, o3,
             sg0, sg1, sg2, sg3, sw0, sw1, sw2, sw3):
    rows = (r0, r1, r2, r3)
    outs = (o0, o1, o2, o3)
    sem_g = (sg0, sg1, sg2, sg3)
    sem_w = (sw0, sw1, sw2, sw3)
    cid = lax.axis_index("c")
    sid = lax.axis_index("s")
    wid = sid * _NC + cid

    pltpu.sync_copy(tok_hbm.at[:, wid], tok_v)
    pltpu.sync_copy(pe_hbm, pe_v)

    @plsc.parallel_loop(0, _ST * 8 * 8, unroll=4)
    def _(j):
        st = j // 64
        r = lax.rem(j, 64)
        s8 = lax.shift_right_logical(r, 3)
        jj = lax.bitwise_and(r, 7)
        sl = pl.ds(16 * jj, 16)
        tok_v[st, s8, sl] = lax.shift_left(tok_v[st, s8, sl], 1)

    iota = lax.iota(jnp.int32, 16)
    dtv = [lax.shift_right_logical(16 * k + iota, 3) for k in range(_KV)]
    d8v = [lax.bitwise_and(16 * k + iota, 7) for k in range(_KV)]

    def fire_gather(c, b):
        st = c // 8
        s8 = lax.rem(c, 8)
        pltpu.async_copy(table_hbm.at[tok_v.at[st, s8]], rows[b], sem_g[b])

    def drain_gather(c, b):
        pltpu.make_async_copy(table_hbm.at[tok_v.at[0, 0]], rows[b],
                              sem_g[b]).wait()

    def out_copy(c, b):
        return pltpu.make_async_copy(outs[b].at[:, :, pl.ds(0, _BT)],
                                     out_hbm.at[c, :, wid], sem_w[b])

    for b in range(_PDIST):
        fire_gather(b, b)

    def group_body(g, carry):
        c0 = g * _NBUF
        for b in range(_NBUF):
            c = c0 + b
            drain_gather(c, b)

            fb = (b + _PDIST) % _NBUF
            if b == 0:
                fire_gather(c + _PDIST, fb)
            else:
                @pl.when(g < _NGROUPS - 1)
                def _():
                    fire_gather(c + _PDIST, fb)

            @pl.when(g >= 1)
            def _():
                out_copy(c - _NBUF, b).wait()

            pe_k = [pe_v[c, pl.ds(16 * k, 16)] for k in range(_KV)]

            @plsc.parallel_loop(0, _BT, unroll=8)
            def _(i):
                colv = jnp.full((16,), i, jnp.int32)
                for k in range(_KV):
                    v = rows[b][i, pl.ds(16 * k, 16)] * _SCALE + pe_k[k]
                    plsc.store_scatter(outs[b], [dtv[k], d8v[k], colv], v)

            out_copy(c, b).start()
        return carry

    lax.fori_loop(0, _NGROUPS, group_body, 0)

    for b in range(_NBUF):
        out_copy(_S - _NBUF + b, b).wait()


@jax.jit
def _music_embedding(tok4, pe_s, table):
    mesh = plsc.VectorSubcoreMesh(
        core_axis_name="c", subcore_axis_name="s",
        num_cores=_NC, num_subcores=_NS,
    )
    run = pl.kernel(
        _sc_body,
        out_type=jax.ShapeDtypeStruct((_S, _D // 8, _NW, 8, _BT), jnp.float32),
        mesh=mesh,
        scratch_types=(
            [pltpu.VMEM((_ST, 8, _BT), jnp.int32),
             pltpu.VMEM((_S, _D), jnp.float32)]
            + [pltpu.VMEM((_BT, _D), jnp.float32) for _ in range(_NBUF)]
            + [pltpu.VMEM((_D // 8, 8, _BT + 1), jnp.float32) for _ in range(_NBUF)]
            + [pltpu.SemaphoreType.DMA for _ in range(2 * _NBUF)]
        ),
        compiler_params=pltpu.CompilerParams(
            use_tc_tiling_on_sc=False, needs_layout_passes=False),
    )
    return run(tok4, pe_s, table)


def kernel(token_ids, table, pe):
    tok4 = (token_ids.astype(jnp.int32).T
            .reshape(_ST, 8, _NW, _BT).transpose(0, 2, 1, 3))
    pe_s = pe[:_S]
    table_p = jnp.pad(table, ((0, 0), (0, _D))).reshape(2 * _VOCAB, _D)
    out5 = _music_embedding(tok4, pe_s, table_p)
    return out5.transpose(2, 4, 0, 1, 3).reshape(_B, _S, _D)


# speedup vs baseline: 1.0088x; 1.0088x over previous
import jax
import jax.numpy as jnp
from jax import lax
from jax.experimental import pallas as pl
from jax.experimental.pallas import tpu as pltpu
from jax.experimental.pallas import tpu_sc as plsc

_VOCAB = 100000
_D = 64
_S = 200
_B = 4096
_NC = 2
_NS = 16
_NW = _NC * _NS
_BT = _B // _NW
_ST = _S // 8
_NBUF = 5
_PDIST = _NBUF - 1
_NGROUPS = _S // _NBUF
_SCALE = 8.0
_KV = _D // 16


def _sc_body(tok_hbm, pe_hbm, table_hbm, out_hbm,
             tok_v, pe_v, r0, r1, r2, r3, r4, o0, o1, o2, o3, o4,
             sg0, sg1, sg2, sg3, sg4, sw0, sw1, sw2, sw3, sw4):
    rows = (r0, r1, r2, r3, r4)
    outs = (o0, o1, o2, o3, o4)
    sem_g = (sg0, sg1, sg2, sg3, sg4)
    sem_w = (sw0, sw1, sw2, sw3, sw4)
    cid = lax.axis_index("c")
    sid = lax.axis_index("s")
    wid = sid * _NC + cid

    pltpu.sync_copy(tok_hbm.at[:, wid], tok_v)
    pltpu.sync_copy(pe_hbm, pe_v)

    @plsc.parallel_loop(0, _ST * 8 * 8, unroll=4)
    def _(j):
        st = j // 64
        r = lax.rem(j, 64)
        s8 = lax.shift_right_logical(r, 3)
        jj = lax.bitwise_and(r, 7)
        sl = pl.ds(16 * jj, 16)
        tok_v[st, s8, sl] = lax.shift_left(tok_v[st, s8, sl], 1)

    iota = lax.iota(jnp.int32, 16)
    dtv = [lax.shift_right_logical(16 * k + iota, 3) for k in range(_KV)]
    d8v = [lax.bitwise_and(16 * k + iota, 7) for k in range(_KV)]

    def fire_gather(c, b):
        st = c // 8
        s8 = lax.rem(c, 8)
        pltpu.async_copy(table_hbm.at[tok_v.at[st, s8]], rows[b], sem_g[b])

    def drain_gather(c, b):
        pltpu.make_async_copy(table_hbm.at[tok_v.at[0, 0]], rows[b],
                              sem_g[b]).wait()

    def out_copy(c, b):
        return pltpu.make_async_copy(outs[b].at[:, :, pl.ds(0, _BT)],
                                     out_hbm.at[c, :, wid], sem_w[b])

    for b in range(_PDIST):
        fire_gather(b, b)

    def group_body(g, carry):
        c0 = g * _NBUF
        for b in range(_NBUF):
            c = c0 + b
            drain_gather(c, b)

            fb = (b + _PDIST) % _NBUF
            if b == 0:
                fire_gather(c + _PDIST, fb)
            else:
                @pl.when(g < _NGROUPS - 1)
                def _():
                    fire_gather(c + _PDIST, fb)

            @pl.when(g >= 1)
            def _():
                out_copy(c - _NBUF, b).wait()

            pe_k = [pe_v[c, pl.ds(16 * k, 16)] for k in range(_KV)]

            @plsc.parallel_loop(0, _BT, unroll=4)
            def _(i):
                colv = jnp.full((16,), i, jnp.int32)
                for k in range(_KV):
                    v = rows[b][i, pl.ds(16 * k, 16)] * _SCALE + pe_k[k]
                    plsc.store_scatter(outs[b], [dtv[k], d8v[k], colv], v)

            out_copy(c, b).start()
        return carry

    lax.fori_loop(0, _NGROUPS, group_body, 0)

    for b in range(_NBUF):
        out_copy(_S - _NBUF + b, b).wait()


@jax.jit
def _music_embedding(tok4, pe_s, table):
    mesh = plsc.VectorSubcoreMesh(
        core_axis_name="c", subcore_axis_name="s",
        num_cores=_NC, num_subcores=_NS,
    )
    run = pl.kernel(
        _sc_body,
        out_type=jax.ShapeDtypeStruct((_S, _D // 8, _NW, 8, _BT), jnp.float32),
        mesh=mesh,
        scratch_types=(
            [pltpu.VMEM((_ST, 8, _BT), jnp.int32),
             pltpu.VMEM((_S, _D), jnp.float32)]
            + [pltpu.VMEM((_BT, _D), jnp.float32) for _ in range(_NBUF)]
            + [pltpu.VMEM((_D // 8, 8, _BT + 1), jnp.float32) for _ in range(_NBUF)]
            + [pltpu.SemaphoreType.DMA for _ in range(2 * _NBUF)]
        ),
        compiler_params=pltpu.CompilerParams(
            use_tc_tiling_on_sc=False, needs_layout_passes=False),
    )
    return run(tok4, pe_s, table)


def kernel(token_ids, table, pe):
    tok4 = (token_ids.astype(jnp.int32).T
            .reshape(_ST, 8, _NW, _BT).transpose(0, 2, 1, 3))
    pe_s = pe[:_S]
    table_p = jnp.pad(table, ((0, 0), (0, _D))).reshape(2 * _VOCAB, _D)
    out5 = _music_embedding(tok4, pe_s, table_p)
    return out5.transpose(2, 4, 0, 1, 3).reshape(_B, _S, _D)
